# baseline (speedup 1.0000x reference)
import jax
import jax.numpy as jnp
from jax.experimental import pallas as pl
from jax.experimental.pallas import tpu as pltpu

_N, _T, _F, _K = 1024, 128, 128, 32
_B = 128
_NB = _N // _B


def _mcc_kernel(x_ref, a_ref, wrel_ref, brel_ref, wroot_ref, wmlp_ref,
                bmlp_ref, s_ref, mc_ref, lo_ref,
                xn_sc, deg_sc, aggr_sc, urs_sc, acc_sc):
    i = pl.program_id(0)

    @pl.when(i == 0)
    def _layernorm():
        x = x_ref[...]
        mu = jnp.mean(x)
        var = jnp.mean((x - mu) ** 2)
        xn_sc[...] = (x - mu) * jax.lax.rsqrt(var + 1e-5)

    @pl.when(i < _NB)
    def _phase1():
        rows = pl.ds(i * _B, _B)
        ab = a_ref[...]
        deg_sc[rows, :] = jnp.sum(ab, axis=1, keepdims=True)
        mask_b = (ab != 0).astype(jnp.float32)
        part = jax.lax.dot_general(mask_b, xn_sc[rows, :],
                                   (((0,), (0,)), ((), ())),
                                   preferred_element_type=jnp.float32)

        @pl.when(i == 0)
        def _():
            aggr_sc[...] = part

        @pl.when(i > 0)
        def _():
            aggr_sc[...] = aggr_sc[...] + part

    @pl.when(i == _NB)
    def _finalize_s():
        rs_col = jax.lax.rsqrt(deg_sc[...])
        w_rel2 = jax.lax.dot_general(wmlp_ref[...], wrel_ref[...],
                                     (((1,), (0,)), ((), ())),
                                     preferred_element_type=jnp.float32)
        w_root2 = jax.lax.dot_general(wmlp_ref[...], wroot_ref[...],
                                      (((1,), (0,)), ((), ())),
                                      preferred_element_type=jnp.float32)
        b2 = jax.lax.dot_general(brel_ref[...], wmlp_ref[...],
                                 (((1,), (1,)), ((), ())),
                                 preferred_element_type=jnp.float32)
        s_logits = (jax.lax.dot_general(aggr_sc[...], w_rel2,
                                        (((1,), (1,)), ((), ())),
                                        preferred_element_type=jnp.float32)
                    + jax.lax.dot_general(xn_sc[...], w_root2,
                                          (((1,), (1,)), ((), ())),
                                          preferred_element_type=jnp.float32)
                    + b2 + bmlp_ref[...])
        s_ref[...] = s_logits
        s = jax.nn.softmax(s_logits, axis=-1)
        u = s * rs_col
        q = jnp.sum(s * s, axis=1, keepdims=True)
        urs_sc[...] = jnp.concatenate([u, rs_col, q], axis=1)

        ss = jax.lax.dot_general(s, s, (((0,), (0,)), ((), ())),
                                 preferred_element_type=jnp.float32)
        n_ss = jnp.sqrt(jnp.sum(ss * ss))
        ii = jax.lax.broadcasted_iota(jnp.int32, (_K, _K), 0)
        jj = jax.lax.broadcasted_iota(jnp.int32, (_K, _K), 1)
        eye = (ii == jj).astype(jnp.float32)
        diff = ss / n_ss - eye / jnp.sqrt(jnp.float32(_K))
        lo_ref[...] = jnp.sqrt(jnp.sum(diff * diff)).reshape(1, 1)

    @pl.when(i >= _NB)
    def _phase2():
        b2i = i - _NB
        rows = pl.ds(b2i * _B, _B)
        ab = a_ref[...]
        a_urs = jax.lax.dot_general(ab, urs_sc[:, : _K + 1],
                                    (((1,), (0,)), ((), ())),
                                    preferred_element_type=jnp.float32)
        u_b = urs_sc[rows, :_K]
        rs_b = urs_sc[rows, _K:_K + 1]
        q_b = urs_sc[rows, _K + 1:_K + 2]
        num_p = jnp.sum(u_b * a_urs[:, :_K])
        den_p = jnp.sum(rs_b * a_urs[:, _K:_K + 1] * q_b)

        @pl.when(i == _NB)
        def _():
            acc_sc[0] = num_p
            acc_sc[1] = den_p

        @pl.when(i > _NB)
        def _():
            acc_sc[0] = acc_sc[0] + num_p
            acc_sc[1] = acc_sc[1] + den_p

    @pl.when(i == 2 * _NB - 1)
    def _finalize_losses():
        mc_ref[...] = jnp.full((1, 1), -(acc_sc[0] / acc_sc[1]),
                               dtype=jnp.float32)


def kernel(X, A, W_rel, b_rel, W_root, W_mlp, b_mlp):
    out_shape = (
        jax.ShapeDtypeStruct((_N, _K), jnp.float32),
        jax.ShapeDtypeStruct((1, 1), jnp.float32),
        jax.ShapeDtypeStruct((1, 1), jnp.float32),
    )
    grid = (2 * _NB,)
    S, mc, lo = pl.pallas_call(
        _mcc_kernel,
        grid=grid,
        in_specs=[
            pl.BlockSpec((_N, _T), lambda i: (0, 0)),
            pl.BlockSpec((_B, _N), lambda i: (i % _NB, 0)),
            pl.BlockSpec((_F, _T), lambda i: (0, 0)),
            pl.BlockSpec((1, _F), lambda i: (0, 0)),
            pl.BlockSpec((_F, _T), lambda i: (0, 0)),
            pl.BlockSpec((_K, _F), lambda i: (0, 0)),
            pl.BlockSpec((1, _K), lambda i: (0, 0)),
        ],
        out_specs=(
            pl.BlockSpec((_N, _K), lambda i: (0, 0)),
            pl.BlockSpec((1, 1), lambda i: (0, 0)),
            pl.BlockSpec((1, 1), lambda i: (0, 0)),
        ),
        scratch_shapes=[
            pltpu.VMEM((_N, _T), jnp.float32),
            pltpu.VMEM((_N, 1), jnp.float32),
            pltpu.VMEM((_N, _T), jnp.float32),
            pltpu.VMEM((_N, _K + 2), jnp.float32),
            pltpu.SMEM((2,), jnp.float32),
        ],
        out_shape=out_shape,
    )(X, A, W_rel, b_rel.reshape(1, _F), W_root, W_mlp, b_mlp.reshape(1, _K))
    return (S, mc[0, 0], lo[0, 0])


# speedup vs baseline: 1.6886x; 1.6886x over previous
import jax
import jax.numpy as jnp
from jax.experimental import pallas as pl
from jax.experimental.pallas import tpu as pltpu

_N, _T, _F, _K = 1024, 128, 128, 32
_B = 128
_NB = _N // _B


def _mcc_kernel(x_ref, a_hbm, wrel_ref, brel_ref, wroot_ref, wmlp_ref,
                bmlp_ref, s_ref, mc_ref, lo_ref, a_vmem, sems):
    copies = [
        pltpu.make_async_copy(
            a_hbm.at[pl.ds(b * _B, _B), :],
            a_vmem.at[pl.ds(b * _B, _B), :],
            sems.at[b],
        )
        for b in range(_NB)
    ]
    for c in copies:
        c.start()

    x = x_ref[...]
    mu = jnp.mean(x)
    var = jnp.mean((x - mu) ** 2)
    xn = (x - mu) * jax.lax.rsqrt(var + 1e-5)

    deg_parts = []
    aggr = jnp.zeros((_N, _T), dtype=jnp.float32)
    for b in range(_NB):
        copies[b].wait()
        ab = a_vmem[pl.ds(b * _B, _B), :]
        deg_parts.append(jnp.sum(ab, axis=1, keepdims=True))
        mask_b = (ab != 0).astype(jnp.float32)
        aggr = aggr + jax.lax.dot_general(
            mask_b, xn[b * _B:(b + 1) * _B, :], (((0,), (0,)), ((), ())),
            preferred_element_type=jnp.float32)
    deg = jnp.concatenate(deg_parts, axis=0)
    rs_col = jax.lax.rsqrt(deg)

    w_rel2 = jax.lax.dot_general(wmlp_ref[...], wrel_ref[...],
                                 (((1,), (0,)), ((), ())),
                                 preferred_element_type=jnp.float32)
    w_root2 = jax.lax.dot_general(wmlp_ref[...], wroot_ref[...],
                                  (((1,), (0,)), ((), ())),
                                  preferred_element_type=jnp.float32)
    b2 = jax.lax.dot_general(brel_ref[...], wmlp_ref[...],
                             (((1,), (1,)), ((), ())),
                             preferred_element_type=jnp.float32)
    s_logits = (jax.lax.dot_general(aggr, w_rel2, (((1,), (1,)), ((), ())),
                                    preferred_element_type=jnp.float32)
                + jax.lax.dot_general(xn, w_root2, (((1,), (1,)), ((), ())),
                                      preferred_element_type=jnp.float32)
                + b2 + bmlp_ref[...])
    s_ref[...] = s_logits

    s = jax.nn.softmax(s_logits, axis=-1)
    u = s * rs_col
    urs = jnp.concatenate([u, rs_col], axis=1)
    a_urs = jax.lax.dot_general(a_vmem[...], urs, (((1,), (0,)), ((), ())),
                                preferred_element_type=jnp.float32)
    au = a_urs[:, :_K]
    d_flat = rs_col * a_urs[:, _K:]
    mincut_num = jnp.sum(u * au)
    mincut_den = jnp.sum(d_flat * jnp.sum(s * s, axis=1, keepdims=True))
    mc_ref[...] = (-(mincut_num / mincut_den)).reshape(1, 1)

    ss = jax.lax.dot_general(s, s, (((0,), (0,)), ((), ())),
                             preferred_element_type=jnp.float32)
    n_ss = jnp.sqrt(jnp.sum(ss * ss))
    ii = jax.lax.broadcasted_iota(jnp.int32, (_K, _K), 0)
    jj = jax.lax.broadcasted_iota(jnp.int32, (_K, _K), 1)
    eye = (ii == jj).astype(jnp.float32)
    diff = ss / n_ss - eye / jnp.sqrt(jnp.float32(_K))
    lo_ref[...] = jnp.sqrt(jnp.sum(diff * diff)).reshape(1, 1)


def kernel(X, A, W_rel, b_rel, W_root, W_mlp, b_mlp):
    out_shape = (
        jax.ShapeDtypeStruct((_N, _K), jnp.float32),
        jax.ShapeDtypeStruct((1, 1), jnp.float32),
        jax.ShapeDtypeStruct((1, 1), jnp.float32),
    )
    S, mc, lo = pl.pallas_call(
        _mcc_kernel,
        in_specs=[
            pl.BlockSpec(memory_space=pltpu.MemorySpace.VMEM),
            pl.BlockSpec(memory_space=pltpu.MemorySpace.HBM),
            pl.BlockSpec(memory_space=pltpu.MemorySpace.VMEM),
            pl.BlockSpec(memory_space=pltpu.MemorySpace.VMEM),
            pl.BlockSpec(memory_space=pltpu.MemorySpace.VMEM),
            pl.BlockSpec(memory_space=pltpu.MemorySpace.VMEM),
            pl.BlockSpec(memory_space=pltpu.MemorySpace.VMEM),
        ],
        out_specs=(
            pl.BlockSpec(memory_space=pltpu.MemorySpace.VMEM),
            pl.BlockSpec(memory_space=pltpu.MemorySpace.VMEM),
            pl.BlockSpec(memory_space=pltpu.MemorySpace.VMEM),
        ),
        scratch_shapes=[
            pltpu.VMEM((_N, _N), jnp.float32),
            pltpu.SemaphoreType.DMA((_NB,)),
        ],
        out_shape=out_shape,
    )(X, A, W_rel, b_rel.reshape(1, _F), W_root, W_mlp, b_mlp.reshape(1, _K))
    return (S, mc[0, 0], lo[0, 0])


# speedup vs baseline: 1.7317x; 1.0255x over previous
import jax
import jax.numpy as jnp
from jax.experimental import pallas as pl
from jax.experimental.pallas import tpu as pltpu

_N, _T, _F, _K = 1024, 128, 128, 32
_B = 512
_NB = _N // _B


def _mcc_kernel(x_ref, a_hbm, wrel_ref, brel_ref, wroot_ref, wmlp_ref,
                bmlp_ref, s_ref, mc_ref, lo_ref, a_vmem, sems):
    copies = [
        pltpu.make_async_copy(
            a_hbm.at[pl.ds(b * _B, _B), :],
            a_vmem.at[pl.ds(b * _B, _B), :],
            sems.at[b],
        )
        for b in range(_NB)
    ]
    for c in copies:
        c.start()

    x = x_ref[...]
    mu = jnp.mean(x)
    var = jnp.mean((x - mu) ** 2)
    xn = (x - mu) * jax.lax.rsqrt(var + 1e-5)

    deg_parts = []
    aggr = jnp.zeros((_N, _T), dtype=jnp.float32)
    for b in range(_NB):
        copies[b].wait()
        ab = a_vmem[pl.ds(b * _B, _B), :]
        deg_parts.append(jnp.sum(ab, axis=1, keepdims=True))
        mask_b = (ab != 0).astype(jnp.float32)
        aggr = aggr + jax.lax.dot_general(
            mask_b, xn[b * _B:(b + 1) * _B, :], (((0,), (0,)), ((), ())),
            preferred_element_type=jnp.float32)
    deg = jnp.concatenate(deg_parts, axis=0)
    rs_col = jax.lax.rsqrt(deg)

    w_rel2 = jax.lax.dot_general(wmlp_ref[...], wrel_ref[...],
                                 (((1,), (0,)), ((), ())),
                                 preferred_element_type=jnp.float32)
    w_root2 = jax.lax.dot_general(wmlp_ref[...], wroot_ref[...],
                                  (((1,), (0,)), ((), ())),
                                  preferred_element_type=jnp.float32)
    b2 = jax.lax.dot_general(brel_ref[...], wmlp_ref[...],
                             (((1,), (1,)), ((), ())),
                             preferred_element_type=jnp.float32)
    s_logits = (jax.lax.dot_general(aggr, w_rel2, (((1,), (1,)), ((), ())),
                                    preferred_element_type=jnp.float32)
                + jax.lax.dot_general(xn, w_root2, (((1,), (1,)), ((), ())),
                                      preferred_element_type=jnp.float32)
                + b2 + bmlp_ref[...])
    s_ref[...] = s_logits

    s = jax.nn.softmax(s_logits, axis=-1)
    u = s * rs_col
    urs = jnp.concatenate([u, rs_col], axis=1)
    a_urs = jax.lax.dot_general(a_vmem[...], urs, (((1,), (0,)), ((), ())),
                                preferred_element_type=jnp.float32)
    au = a_urs[:, :_K]
    d_flat = rs_col * a_urs[:, _K:]
    mincut_num = jnp.sum(u * au)
    mincut_den = jnp.sum(d_flat * jnp.sum(s * s, axis=1, keepdims=True))
    mc_ref[...] = (-(mincut_num / mincut_den)).reshape(1, 1)

    ss = jax.lax.dot_general(s, s, (((0,), (0,)), ((), ())),
                             preferred_element_type=jnp.float32)
    n_ss = jnp.sqrt(jnp.sum(ss * ss))
    ii = jax.lax.broadcasted_iota(jnp.int32, (_K, _K), 0)
    jj = jax.lax.broadcasted_iota(jnp.int32, (_K, _K), 1)
    eye = (ii == jj).astype(jnp.float32)
    diff = ss / n_ss - eye / jnp.sqrt(jnp.float32(_K))
    lo_ref[...] = jnp.sqrt(jnp.sum(diff * diff)).reshape(1, 1)


def kernel(X, A, W_rel, b_rel, W_root, W_mlp, b_mlp):
    out_shape = (
        jax.ShapeDtypeStruct((_N, _K), jnp.float32),
        jax.ShapeDtypeStruct((1, 1), jnp.float32),
        jax.ShapeDtypeStruct((1, 1), jnp.float32),
    )
    S, mc, lo = pl.pallas_call(
        _mcc_kernel,
        in_specs=[
            pl.BlockSpec(memory_space=pltpu.MemorySpace.VMEM),
            pl.BlockSpec(memory_space=pltpu.MemorySpace.HBM),
            pl.BlockSpec(memory_space=pltpu.MemorySpace.VMEM),
            pl.BlockSpec(memory_space=pltpu.MemorySpace.VMEM),
            pl.BlockSpec(memory_space=pltpu.MemorySpace.VMEM),
            pl.BlockSpec(memory_space=pltpu.MemorySpace.VMEM),
            pl.BlockSpec(memory_space=pltpu.MemorySpace.VMEM),
        ],
        out_specs=(
            pl.BlockSpec(memory_space=pltpu.MemorySpace.VMEM),
            pl.BlockSpec(memory_space=pltpu.MemorySpace.VMEM),
            pl.BlockSpec(memory_space=pltpu.MemorySpace.VMEM),
        ),
        scratch_shapes=[
            pltpu.VMEM((_N, _N), jnp.float32),
            pltpu.SemaphoreType.DMA((_NB,)),
        ],
        out_shape=out_shape,
    )(X, A, W_rel, b_rel.reshape(1, _F), W_root, W_mlp, b_mlp.reshape(1, _K))
    return (S, mc[0, 0], lo[0, 0])
